# baseline (speedup 1.0000x reference)
import jax
import jax.numpy as jnp
from jax.experimental import pallas as pl
from jax.experimental.pallas import tpu as pltpu

_LANES = 16384


def _body(t_ref, rot_ref, sc_ref, op_ref, xyz_ref, vel_ref, to_ref,
          fdc_ref, fr_ref,
          xyzt_ref, rotn_ref, scale_ref, opac_ref, feats_ref):
    r = rot_ref[...]
    s = jnp.sum(r * r, axis=0, keepdims=True)
    inv = jax.lax.rsqrt(jnp.maximum(s, 1e-24))
    rotn_ref[...] = r * inv

    scale_ref[...] = jnp.exp(sc_ref[...])
    opac_ref[...] = jax.nn.sigmoid(op_ref[...])

    dt = t_ref[0] - to_ref[...]
    xyzt_ref[...] = xyz_ref[...] + vel_ref[...] * dt

    feats_ref[:, 0, :] = fdc_ref[:, 0, :]
    feats_ref[:, 1:16, :] = fr_ref[...]


def kernel(xyz, rotation, scaling, opacity, features_dc, features_rest, time_offset, velocity, time):
    n = xyz.shape[0]
    t = jnp.asarray(time, jnp.float32).reshape(1)

    rot_t = rotation.T
    sc_t = scaling.T
    op_t = opacity.T
    xyz_t_in = xyz.T
    vel_t = velocity.T
    to_t = time_offset.T
    fdc_t = jnp.transpose(features_dc, (2, 1, 0))
    fr_t = jnp.transpose(features_rest, (2, 1, 0))

    L = _LANES
    g = pl.cdiv(n, L)

    def cols(d):
        return pl.BlockSpec((d, L), lambda i: (0, i))

    in_specs = [
        pl.BlockSpec(memory_space=pltpu.SMEM),
        cols(4),
        cols(3),
        cols(1),
        cols(3),
        cols(3),
        cols(1),
        pl.BlockSpec((3, 1, L), lambda i: (0, 0, i)),
        pl.BlockSpec((3, 15, L), lambda i: (0, 0, i)),
    ]
    out_specs = [
        cols(3),
        cols(4),
        cols(3),
        cols(1),
        pl.BlockSpec((3, 16, L), lambda i: (0, 0, i)),
    ]
    out_shape = [
        jax.ShapeDtypeStruct((3, n), jnp.float32),
        jax.ShapeDtypeStruct((4, n), jnp.float32),
        jax.ShapeDtypeStruct((3, n), jnp.float32),
        jax.ShapeDtypeStruct((1, n), jnp.float32),
        jax.ShapeDtypeStruct((3, 16, n), jnp.float32),
    ]
    xyzt_T, rotn_T, scale_T, opac_T, feats_T = pl.pallas_call(
        _body,
        grid=(g,),
        in_specs=in_specs,
        out_specs=out_specs,
        out_shape=out_shape,
        compiler_params=pltpu.CompilerParams(
            dimension_semantics=("arbitrary",),
        ),
    )(t, rot_t, sc_t, op_t, xyz_t_in, vel_t, to_t, fdc_t, fr_t)
    return (
        xyzt_T.T,
        rotn_T.T,
        scale_T.T,
        opac_T.T,
        jnp.transpose(feats_T, (2, 1, 0)),
    )


# speedup vs baseline: 1.0360x; 1.0360x over previous
import jax
import jax.numpy as jnp
from jax.experimental import pallas as pl
from jax.experimental.pallas import tpu as pltpu

_LANES = 57344


def _body(t_ref, rot_ref, sc_ref, op_ref, xyz_ref, vel_ref, to_ref,
          fdc_ref, fr_ref,
          xyzt_ref, rotn_ref, scale_ref, opac_ref, feats_ref):
    r = rot_ref[...]
    s = jnp.sum(r * r, axis=0, keepdims=True)
    inv = jax.lax.rsqrt(jnp.maximum(s, 1e-24))
    rotn_ref[...] = r * inv

    scale_ref[...] = jnp.exp(sc_ref[...])
    opac_ref[...] = jax.nn.sigmoid(op_ref[...])

    dt = t_ref[0] - to_ref[...]
    xyzt_ref[...] = xyz_ref[...] + vel_ref[...] * dt

    feats_ref[:, 0, :] = fdc_ref[:, 0, :]
    feats_ref[:, 1:16, :] = fr_ref[...]


def kernel(xyz, rotation, scaling, opacity, features_dc, features_rest, time_offset, velocity, time):
    n = xyz.shape[0]
    t = jnp.asarray(time, jnp.float32).reshape(1)

    rot_t = rotation.T
    sc_t = scaling.T
    op_t = opacity.T
    xyz_t_in = xyz.T
    vel_t = velocity.T
    to_t = time_offset.T
    fdc_t = jnp.transpose(features_dc, (2, 1, 0))
    fr_t = jnp.transpose(features_rest, (2, 1, 0))

    L = _LANES
    g = pl.cdiv(n, L)

    def cols(d):
        return pl.BlockSpec((d, L), lambda i: (0, i))

    in_specs = [
        pl.BlockSpec(memory_space=pltpu.SMEM),
        cols(4),
        cols(3),
        cols(1),
        cols(3),
        cols(3),
        cols(1),
        pl.BlockSpec((3, 1, L), lambda i: (0, 0, i)),
        pl.BlockSpec((3, 15, L), lambda i: (0, 0, i)),
    ]
    out_specs = [
        cols(3),
        cols(4),
        cols(3),
        cols(1),
        pl.BlockSpec((3, 16, L), lambda i: (0, 0, i)),
    ]
    out_shape = [
        jax.ShapeDtypeStruct((3, n), jnp.float32),
        jax.ShapeDtypeStruct((4, n), jnp.float32),
        jax.ShapeDtypeStruct((3, n), jnp.float32),
        jax.ShapeDtypeStruct((1, n), jnp.float32),
        jax.ShapeDtypeStruct((3, 16, n), jnp.float32),
    ]
    xyzt_T, rotn_T, scale_T, opac_T, feats_T = pl.pallas_call(
        _body,
        grid=(g,),
        in_specs=in_specs,
        out_specs=out_specs,
        out_shape=out_shape,
        compiler_params=pltpu.CompilerParams(
            dimension_semantics=("arbitrary",),
        ),
    )(t, rot_t, sc_t, op_t, xyz_t_in, vel_t, to_t, fdc_t, fr_t)
    return (
        xyzt_T.T,
        rotn_T.T,
        scale_T.T,
        opac_T.T,
        jnp.transpose(feats_T, (2, 1, 0)),
    )
